# Initial kernel scaffold; baseline (speedup 1.0000x reference)
import jax
import jax.numpy as jnp
from jax.experimental import pallas as pl


def kernel(X, pro_id, celeb_id, season, beta, u_pro, v_celeb, w_season):
    raise NotImplementedError("write your pallas kernel here")


# speedup vs baseline: 2.7366x; 2.7366x over previous
import functools

import jax
import jax.numpy as jnp
from jax import lax
from jax.experimental import pallas as pl
from jax.experimental.pallas import tpu as pltpu
from jax.experimental.pallas import tpu_sc as plsc

N = 16384
D = 64

_NC = 2
_NS = 16
_NW = _NC * _NS
_RPW = N // _NW
_CHUNK = 128
_NCH = _RPW // _CHUNK


def _matvec_body(x_ref, b_ref, o_ref):
    o_ref[0, 0, :] = jnp.sum(x_ref[0] * b_ref[0][None, :], axis=1)


def _matvec(X, beta):
    out = pl.pallas_call(
        _matvec_body,
        grid=(8,),
        in_specs=[
            pl.BlockSpec((1, N // 8, D), lambda i: (i, 0, 0)),
            pl.BlockSpec((1, D), lambda i: (0, 0)),
        ],
        out_specs=pl.BlockSpec((1, 1, N // 8), lambda i: (i, 0, 0)),
        out_shape=jax.ShapeDtypeStruct((8, 1, N // 8), jnp.float32),
    )(X.reshape(8, N // 8, D), beta.reshape(1, D))
    return out.reshape(N)


_mesh = plsc.VectorSubcoreMesh(core_axis_name="c", subcore_axis_name="s")


@functools.partial(
    pl.kernel,
    mesh=_mesh,
    out_type=jax.ShapeDtypeStruct((N,), jnp.float32),
    scratch_types=[
        pltpu.VMEM((_RPW,), jnp.int32),
        pltpu.VMEM((_RPW,), jnp.int32),
        pltpu.VMEM((_RPW,), jnp.int32),
        pltpu.VMEM((_RPW,), jnp.float32),
        pltpu.VMEM((_RPW,), jnp.float32),
        pltpu.VMEM((_RPW,), jnp.float32),
        pltpu.VMEM((_RPW,), jnp.float32),
        pltpu.SemaphoreType.DMA,
        pltpu.SemaphoreType.DMA,
    ],
)
def _sc_gathersum(xb_hbm, pro_hbm, celeb_hbm, season_hbm, u_hbm, v_hbm, w_hbm,
                  out_hbm, idu, idv, ids, xbv, gu, gv, gw, sem_i, sem_g):
    wid = lax.axis_index("s") * _NC + lax.axis_index("c")
    base = wid * _RPW

    stage = [
        pltpu.async_copy(pro_hbm.at[pl.ds(base, _RPW)], idu, sem_i),
        pltpu.async_copy(celeb_hbm.at[pl.ds(base, _RPW)], idv, sem_i),
        pltpu.async_copy(season_hbm.at[pl.ds(base, _RPW)], ids, sem_i),
        pltpu.async_copy(xb_hbm.at[pl.ds(base, _RPW)], xbv, sem_i),
    ]
    for c in stage:
        c.wait()

    gathers = []
    for j in range(_NCH):
        sl = pl.ds(j * _CHUNK, _CHUNK)
        gathers.append(pltpu.async_copy(u_hbm.at[idu.at[sl]], gu.at[sl], sem_g))
        gathers.append(pltpu.async_copy(v_hbm.at[idv.at[sl]], gv.at[sl], sem_g))
        gathers.append(pltpu.async_copy(w_hbm.at[ids.at[sl]], gw.at[sl], sem_g))
    for c in gathers:
        c.wait()

    for i in range(_RPW // 16):
        s = pl.ds(i * 16, 16)
        xbv[s] = xbv[s] + gu[s] + gv[s] + gw[s]

    pltpu.sync_copy(xbv, out_hbm.at[pl.ds(base, _RPW)])


def kernel(X, pro_id, celeb_id, season, beta, u_pro, v_celeb, w_season):
    xb = _matvec(X, beta)
    return _sc_gathersum(
        xb,
        pro_id.astype(jnp.int32),
        celeb_id.astype(jnp.int32),
        season.astype(jnp.int32),
        u_pro,
        v_celeb,
        w_season,
    )


# speedup vs baseline: 3.8113x; 1.3927x over previous
import functools

import jax
import jax.numpy as jnp
from jax import lax
from jax.experimental import pallas as pl
from jax.experimental.pallas import tpu as pltpu
from jax.experimental.pallas import tpu_sc as plsc

N = 16384
D = 64

_NC = 2
_NS = 16
_NW = _NC * _NS
_RPW = N // _NW
_CHUNK = 128
_NCH = _RPW // _CHUNK

_mesh = plsc.VectorSubcoreMesh(core_axis_name="c", subcore_axis_name="s")


@functools.partial(
    pl.kernel,
    mesh=_mesh,
    out_type=jax.ShapeDtypeStruct((N,), jnp.float32),
    scratch_types=[
        pltpu.VMEM((_RPW,), jnp.int32),
        pltpu.VMEM((_RPW,), jnp.int32),
        pltpu.VMEM((_RPW,), jnp.int32),
        pltpu.VMEM((D, _RPW), jnp.float32),
        pltpu.VMEM((D * 16,), jnp.float32),
        pltpu.VMEM((_RPW,), jnp.float32),
        pltpu.VMEM((_RPW,), jnp.float32),
        pltpu.VMEM((_RPW,), jnp.float32),
        pltpu.VMEM((_RPW,), jnp.float32),
        pltpu.SemaphoreType.DMA,
        pltpu.SemaphoreType.DMA,
        pltpu.SemaphoreType.DMA,
    ],
)
def _sc_fused(xt_hbm, pro_hbm, celeb_hbm, season_hbm, beta_hbm, u_hbm, v_hbm,
              w_hbm, out_hbm, idu, idv, ids, xcol, bbv, acc, gu, gv, gw,
              sem_i, sem_x, sem_g):
    wid = lax.axis_index("s") * _NC + lax.axis_index("c")
    base = wid * _RPW

    stage = [
        pltpu.async_copy(pro_hbm.at[pl.ds(base, _RPW)], idu, sem_i),
        pltpu.async_copy(celeb_hbm.at[pl.ds(base, _RPW)], idv, sem_i),
        pltpu.async_copy(season_hbm.at[pl.ds(base, _RPW)], ids, sem_i),
        pltpu.async_copy(beta_hbm, bbv, sem_i),
    ]
    xcp = pltpu.async_copy(xt_hbm.at[:, pl.ds(base, _RPW)], xcol, sem_x)
    for c in stage:
        c.wait()

    gathers = []
    for j in range(_NCH):
        sl = pl.ds(j * _CHUNK, _CHUNK)
        gathers.append(pltpu.async_copy(u_hbm.at[idu.at[sl]], gu.at[sl], sem_g))
        gathers.append(pltpu.async_copy(v_hbm.at[idv.at[sl]], gv.at[sl], sem_g))
        gathers.append(pltpu.async_copy(w_hbm.at[ids.at[sl]], gw.at[sl], sem_g))

    xcp.wait()

    def chunk_body(c, _):
        r = pl.ds(c * 16, 16)
        a = xcol[0, r] * bbv[pl.ds(0, 16)]
        for d in range(1, D):
            a = a + xcol[d, r] * bbv[pl.ds(d * 16, 16)]
        acc[r] = a
        return _

    lax.fori_loop(0, _RPW // 16, chunk_body, 0)

    for c in gathers:
        c.wait()

    for i in range(_RPW // 16):
        s = pl.ds(i * 16, 16)
        acc[s] = acc[s] + gu[s] + gv[s] + gw[s]

    pltpu.sync_copy(acc, out_hbm.at[pl.ds(base, _RPW)])


def kernel(X, pro_id, celeb_id, season, beta, u_pro, v_celeb, w_season):
    return _sc_fused(
        X.T,
        pro_id.astype(jnp.int32),
        celeb_id.astype(jnp.int32),
        season.astype(jnp.int32),
        jnp.repeat(beta, 16),
        u_pro,
        v_celeb,
        w_season,
    )


# speedup vs baseline: 4.7114x; 1.2362x over previous
import functools

import jax
import jax.numpy as jnp
from jax import lax
from jax.experimental import pallas as pl
from jax.experimental.pallas import tpu as pltpu
from jax.experimental.pallas import tpu_sc as plsc

N = 16384
D = 64

_NC = 2
_NS = 16
_NW = _NC * _NS
_RPW = N // _NW
_CHUNK = 128
_NCH = _RPW // _CHUNK

_mesh = plsc.VectorSubcoreMesh(core_axis_name="c", subcore_axis_name="s")


@functools.partial(
    pl.kernel,
    mesh=_mesh,
    compiler_params=pltpu.CompilerParams(needs_layout_passes=False),
    out_type=jax.ShapeDtypeStruct((N,), jnp.float32),
    scratch_types=[
        pltpu.VMEM((_RPW,), jnp.int32),
        pltpu.VMEM((_RPW,), jnp.int32),
        pltpu.VMEM((_RPW,), jnp.int32),
        pltpu.VMEM((D, _RPW), jnp.float32),
        pltpu.VMEM((D * 16,), jnp.float32),
        pltpu.VMEM((_RPW,), jnp.float32),
        pltpu.VMEM((_RPW,), jnp.float32),
        pltpu.VMEM((_RPW,), jnp.float32),
        pltpu.VMEM((1024,), jnp.float32),
        pltpu.SemaphoreType.DMA,
        pltpu.SemaphoreType.DMA,
        pltpu.SemaphoreType.DMA,
    ],
)
def _sc_fused(xt_hbm, pro_hbm, celeb_hbm, season_hbm, beta_hbm, u_hbm, v_hbm,
              w_hbm, out_hbm, idu, idv, ids, xcol, bbv, acc, gu, gv, wtab,
              sem_i, sem_x, sem_g):
    wid = lax.axis_index("s") * _NC + lax.axis_index("c")
    base = wid * _RPW

    stage = [
        pltpu.async_copy(pro_hbm.at[pl.ds(base, _RPW)], idu, sem_i),
        pltpu.async_copy(celeb_hbm.at[pl.ds(base, _RPW)], idv, sem_i),
        pltpu.async_copy(season_hbm.at[pl.ds(base, _RPW)], ids, sem_i),
        pltpu.async_copy(beta_hbm, bbv, sem_i),
        pltpu.async_copy(w_hbm, wtab.at[pl.ds(0, 1000)], sem_i),
    ]
    xcp = pltpu.async_copy(xt_hbm.at[:, pl.ds(base, _RPW)], xcol, sem_x)
    for c in stage:
        c.wait()

    gathers = []
    for j in range(_NCH):
        sl = pl.ds(j * _CHUNK, _CHUNK)
        gathers.append(pltpu.async_copy(u_hbm.at[idu.at[sl]], gu.at[sl], sem_g))
        gathers.append(pltpu.async_copy(v_hbm.at[idv.at[sl]], gv.at[sl], sem_g))

    xcp.wait()

    def chunk_body(c, _):
        r = pl.ds(c * 16, 16)
        a = xcol[0, r] * bbv[pl.ds(0, 16)]
        for d in range(1, D):
            a = a + xcol[d, r] * bbv[pl.ds(d * 16, 16)]
        acc[r] = a
        return _

    lax.fori_loop(0, _RPW // 16, chunk_body, 0)

    for i in range(_RPW // 16):
        s = pl.ds(i * 16, 16)
        acc[s] = acc[s] + plsc.load_gather(wtab, [ids[s]])

    for c in gathers:
        c.wait()

    for i in range(_RPW // 16):
        s = pl.ds(i * 16, 16)
        acc[s] = acc[s] + gu[s] + gv[s]

    pltpu.sync_copy(acc, out_hbm.at[pl.ds(base, _RPW)])


def kernel(X, pro_id, celeb_id, season, beta, u_pro, v_celeb, w_season):
    return _sc_fused(
        X.T,
        pro_id.astype(jnp.int32),
        celeb_id.astype(jnp.int32),
        season.astype(jnp.int32),
        jnp.repeat(beta, 16),
        u_pro,
        v_celeb,
        w_season,
    )


# speedup vs baseline: 4.8413x; 1.0276x over previous
import functools

import jax
import jax.numpy as jnp
from jax import lax
from jax.experimental import pallas as pl
from jax.experimental.pallas import tpu as pltpu
from jax.experimental.pallas import tpu_sc as plsc

N = 16384
D = 64

_NC = 2
_NS = 16
_NW = _NC * _NS
_RPW = N // _NW
_CHUNK = 64
_NCH = _RPW // _CHUNK

_mesh = plsc.VectorSubcoreMesh(core_axis_name="c", subcore_axis_name="s")


@functools.partial(
    pl.kernel,
    mesh=_mesh,
    compiler_params=pltpu.CompilerParams(needs_layout_passes=False),
    out_type=jax.ShapeDtypeStruct((N,), jnp.float32),
    scratch_types=[
        pltpu.VMEM((_RPW,), jnp.int32),
        pltpu.VMEM((_RPW,), jnp.int32),
        pltpu.VMEM((_RPW,), jnp.int32),
        pltpu.VMEM((D, _RPW), jnp.float32),
        pltpu.VMEM((D * 16,), jnp.float32),
        pltpu.VMEM((_RPW,), jnp.float32),
        pltpu.VMEM((_RPW,), jnp.float32),
        pltpu.VMEM((_RPW,), jnp.float32),
        pltpu.VMEM((1024,), jnp.float32),
        pltpu.SemaphoreType.DMA,
        pltpu.SemaphoreType.DMA,
        pltpu.SemaphoreType.DMA,
    ],
)
def _sc_fused(xt_hbm, pro_hbm, celeb_hbm, season_hbm, beta_hbm, u_hbm, v_hbm,
              w_hbm, out_hbm, idu, idv, ids, xcol, bbv, acc, gu, gv, wtab,
              sem_i, sem_x, sem_g):
    wid = lax.axis_index("s") * _NC + lax.axis_index("c")
    base = wid * _RPW

    stage = [
        pltpu.async_copy(pro_hbm.at[pl.ds(base, _RPW)], idu, sem_i),
        pltpu.async_copy(celeb_hbm.at[pl.ds(base, _RPW)], idv, sem_i),
        pltpu.async_copy(season_hbm.at[pl.ds(base, _RPW)], ids, sem_i),
        pltpu.async_copy(beta_hbm, bbv, sem_i),
        pltpu.async_copy(w_hbm, wtab.at[pl.ds(0, 1000)], sem_i),
    ]
    xcp = pltpu.async_copy(xt_hbm.at[:, pl.ds(base, _RPW)], xcol, sem_x)
    for c in stage:
        c.wait()

    gathers = []
    for j in range(_NCH):
        sl = pl.ds(j * _CHUNK, _CHUNK)
        gathers.append(pltpu.async_copy(u_hbm.at[idu.at[sl]], gu.at[sl], sem_g))
        gathers.append(pltpu.async_copy(v_hbm.at[idv.at[sl]], gv.at[sl], sem_g))

    xcp.wait()

    def chunk_body(c, _):
        r = pl.ds(c * 16, 16)
        a = xcol[0, r] * bbv[pl.ds(0, 16)]
        for d in range(1, D):
            a = a + xcol[d, r] * bbv[pl.ds(d * 16, 16)]
        acc[r] = a
        return _

    lax.fori_loop(0, _RPW // 16, chunk_body, 0)

    for i in range(_RPW // 16):
        s = pl.ds(i * 16, 16)
        acc[s] = acc[s] + plsc.load_gather(wtab, [ids[s]])

    for c in gathers:
        c.wait()

    for i in range(_RPW // 16):
        s = pl.ds(i * 16, 16)
        acc[s] = acc[s] + gu[s] + gv[s]

    pltpu.sync_copy(acc, out_hbm.at[pl.ds(base, _RPW)])


def kernel(X, pro_id, celeb_id, season, beta, u_pro, v_celeb, w_season):
    return _sc_fused(
        X.T,
        pro_id.astype(jnp.int32),
        celeb_id.astype(jnp.int32),
        season.astype(jnp.int32),
        jnp.repeat(beta, 16),
        u_pro,
        v_celeb,
        w_season,
    )
